# Initial kernel scaffold; baseline (speedup 1.0000x reference)
import jax
import jax.numpy as jnp
from jax.experimental import pallas as pl


def kernel(pos, batch, mlp1, mlp2, mlp3, head):
    raise NotImplementedError("write your pallas kernel here")


# speedup vs baseline: 15.2700x; 15.2700x over previous
import functools

import jax
import jax.numpy as jnp
from jax import lax
from jax.experimental import pallas as pl
from jax.experimental.pallas import tpu as pltpu
from jax.experimental.pallas import tpu_sc as plsc

NPTS = 16384
NCLOUD = 16
N1 = NPTS // 2 + NCLOUD
N2 = NPTS // 8 + NCLOUD
N1P = 8704
N2P = 2304
MAXNN = 64
SLOT = 80
E2P = N2P * MAXNN
NW = 32

_F32 = jnp.float32
_I32 = jnp.int32


def _dot(a, b):
    return jax.lax.dot_general(
        a, b, (((1,), (0,)), ((), ())),
        preferred_element_type=_F32)


def _make_search(np_, rw, ncand, r2, emit_nbr):
    mesh = plsc.VectorSubcoreMesh(core_axis_name="c", subcore_axis_name="s")
    outs = [jax.ShapeDtypeStruct((np_, SLOT), _F32) for _ in range(3)]
    if emit_nbr:
        outs.append(jax.ShapeDtypeStruct((np_, SLOT), _I32))
    outs.append(jax.ShapeDtypeStruct((np_,), _I32))
    scratch = [pltpu.VMEM((ncand + 16,), _F32) for _ in range(3)]
    scratch += [pltpu.VMEM((rw + 16,), _F32) for _ in range(3)]
    scratch += [pltpu.VMEM((rw + 16,), _I32) for _ in range(2)]
    scratch += [pltpu.VMEM((SLOT,), _F32) for _ in range(3)]
    if emit_nbr:
        scratch.append(pltpu.VMEM((SLOT,), _I32))
    scratch.append(pltpu.VMEM((rw,), _I32))

    @functools.partial(
        pl.kernel, out_type=tuple(outs), mesh=mesh,
        compiler_params=pltpu.CompilerParams(needs_layout_passes=False),
        scratch_types=tuple(scratch))
    def k(*refs):
        it = iter(refs)
        px, py, pz, cx, cy, cz, cst, cln = (next(it) for _ in range(8))
        odx, ody, odz = (next(it) for _ in range(3))
        onb = next(it) if emit_nbr else None
        ocnt = next(it)
        pxv, pyv, pzv = (next(it) for _ in range(3))
        cxv, cyv, czv = (next(it) for _ in range(3))
        cstv, clnv = (next(it) for _ in range(2))
        bdx, bdy, bdz = (next(it) for _ in range(3))
        bnb = next(it) if emit_nbr else None
        bcv = next(it)

        w = lax.axis_index("c") * 16 + lax.axis_index("s")
        base = w * rw
        pltpu.sync_copy(px, pxv.at[pl.ds(0, ncand)])
        pltpu.sync_copy(py, pyv.at[pl.ds(0, ncand)])
        pltpu.sync_copy(pz, pzv.at[pl.ds(0, ncand)])
        pltpu.sync_copy(cx.at[pl.ds(base, rw)], cxv.at[pl.ds(0, rw)])
        pltpu.sync_copy(cy.at[pl.ds(base, rw)], cyv.at[pl.ds(0, rw)])
        pltpu.sync_copy(cz.at[pl.ds(base, rw)], czv.at[pl.ds(0, rw)])
        pltpu.sync_copy(cst.at[pl.ds(base, rw)], cstv.at[pl.ds(0, rw)])
        pltpu.sync_copy(cln.at[pl.ds(base, rw)], clnv.at[pl.ds(0, rw)])

        zf = jnp.zeros((16,), _F32)
        zi = jnp.zeros((16,), _I32)
        lane = lax.iota(_I32, 16)

        def per_cent(j, carry):
            cxs = cxv[pl.ds(j, 16)][0]
            cys = cyv[pl.ds(j, 16)][0]
            czs = czv[pl.ds(j, 16)][0]
            s = cstv[pl.ds(j, 16)][0]
            n = clnv[pl.ds(j, 16)][0]
            for t in range(SLOT // 16):
                bdx[pl.ds(t * 16, 16)] = zf
                bdy[pl.ds(t * 16, 16)] = zf
                bdz[pl.ds(t * 16, 16)] = zf
                if emit_nbr:
                    bnb[pl.ds(t * 16, 16)] = zi

            nch = (n + 15) // 16

            @plsc.parallel_loop(0, nch, carry=jnp.int32(0))
            def cnt(t, c):
                o = t * 16
                off = s + o
                X = pxv[pl.ds(off, 16)]
                Y = pyv[pl.ds(off, 16)]
                Z = pzv[pl.ds(off, 16)]
                dx = X - cxs
                dy = Y - cys
                dz = Z - czs
                d2 = dx * dx + dy * dy + dz * dz
                m = jnp.logical_and(d2 <= r2, (o + lane) < n)
                pf = jnp.cumsum(m.astype(_I32))
                m2 = jnp.logical_and(m, pf <= (MAXNN - c))
                add = jnp.max(jnp.where(m2, pf, 0))
                plsc.store_compressed(bdx.at[pl.ds(c, 16)], dx, mask=m2)
                plsc.store_compressed(bdy.at[pl.ds(c, 16)], dy, mask=m2)
                plsc.store_compressed(bdz.at[pl.ds(c, 16)], dz, mask=m2)
                if emit_nbr:
                    plsc.store_compressed(bnb.at[pl.ds(c, 16)], off + lane,
                                          mask=m2)
                return c + add
            plsc.store_scatter(bcv, [jnp.full((16,), j, _I32)],
                               jnp.full((16,), cnt, _I32),
                               mask=lane == 0)
            gi = base + j
            pltpu.sync_copy(bdx, odx.at[gi])
            pltpu.sync_copy(bdy, ody.at[gi])
            pltpu.sync_copy(bdz, odz.at[gi])
            if emit_nbr:
                pltpu.sync_copy(bnb, onb.at[gi])
            return carry

        lax.fori_loop(0, rw, per_cent, 0)
        pltpu.sync_copy(bcv, ocnt.at[pl.ds(base, rw)])

    return k


def _search_l1(px, py, pz, cx, cy, cz, cst, cln):
    f = _make_search(N1P, N1P // NW, NPTS, 0.2 * 0.2, False)
    return f(px, py, pz, cx, cy, cz, cst, cln)


def _search_l2(px, py, pz, cx, cy, cz, cst, cln):
    f = _make_search(N2P, N2P // NW, N1P, 0.4 * 0.4, True)
    return f(px, py, pz, cx, cy, cz, cst, cln)


def _gather_rows(tab, idx):
    mesh = plsc.VectorSubcoreMesh(core_axis_name="c", subcore_axis_name="s")
    per_w = E2P // NW
    nchunk = per_w // 128

    @functools.partial(
        pl.kernel, mesh=mesh,
        out_type=jax.ShapeDtypeStruct((E2P, 128), _F32),
        compiler_params=pltpu.CompilerParams(needs_layout_passes=False),
        scratch_types=(pltpu.VMEM((128,), _I32),
                       pltpu.VMEM((128, 128), _F32),
                       pltpu.SemaphoreType.DMA))
    def g(tabh, idxh, outh, idxv, rowsv, sem):
        w = lax.axis_index("c") * 16 + lax.axis_index("s")

        def chunk(c, carry):
            b = w * per_w + c * 128
            pltpu.sync_copy(idxh.at[pl.ds(b, 128)], idxv)
            pltpu.async_copy(tabh.at[idxv], rowsv, sem).wait()
            pltpu.sync_copy(rowsv, outh.at[pl.ds(b, 128)])
            return carry

        lax.fori_loop(0, nchunk, chunk, 0)

    return g(tab, idx)


def _acc_stats(outr, z, vm, step):
    S = (z * vm).sum(0)
    Q = (z * z * vm).sum(0)
    n = vm.sum()
    d = z.shape[1]
    Sp = jnp.broadcast_to(jnp.pad(S, (0, 128 - d)), (8, 128))
    Qp = jnp.broadcast_to(jnp.pad(Q, (0, 128 - d)), (8, 128))
    row = lax.broadcasted_iota(_I32, (8, 128), 0)
    col = lax.broadcasted_iota(_I32, (8, 128), 1)
    acc = (jnp.where(row == 0, Sp, 0.0) + jnp.where(row == 1, Qp, 0.0)
           + jnp.where(jnp.logical_and(row == 2, col == 0), n, 0.0))

    @pl.when(step == 0)
    def _():
        outr[...] = jnp.zeros_like(outr)

    outr[...] += acc


def _e_spec(R, d):
    return pl.BlockSpec((R, d), lambda i: (i, 0))


def _w_spec(a, b):
    return pl.BlockSpec((a, b), lambda i: (0, 0))


def _stats1(msg8, w8, R, F, xg=None, wx=None):
    has_x = xg is not None

    def body(*refs):
        if has_x:
            m8r, xgr, w8r, wxr, outr = refs
            z = _dot(m8r[...], w8r[...]) + _dot(xgr[...], wxr[...])
        else:
            m8r, w8r, outr = refs
            z = _dot(m8r[...], w8r[...])
        _acc_stats(outr, z, m8r[...][:, 4:5], pl.program_id(0))

    E = msg8.shape[0]
    ins = [msg8] + ([xg] if has_x else []) + [w8] + ([wx] if has_x else [])
    specs = [_e_spec(R, 8)] + ([_e_spec(R, 128)] if has_x else [])         + [_w_spec(8, F)] + ([_w_spec(128, F)] if has_x else [])
    return pl.pallas_call(
        body, grid=(E // R,), in_specs=specs,
        out_specs=pl.BlockSpec((8, 128), lambda i: (0, 0)),
        out_shape=jax.ShapeDtypeStruct((8, 128), _F32),
    )(*ins)


def _stats2(msg8, w8, p1, w2, b2, R, F, G, xg=None, wx=None):
    has_x = xg is not None

    def body(*refs):
        if has_x:
            m8r, xgr, w8r, wxr, p1r, w2r, b2r, outr = refs
            z1 = _dot(m8r[...], w8r[...]) + _dot(xgr[...], wxr[...])
        else:
            m8r, w8r, p1r, w2r, b2r, outr = refs
            z1 = _dot(m8r[...], w8r[...])
        p1v = p1r[...]
        a1 = jax.nn.relu(z1 * p1v[0:1] + p1v[1:2])
        z2 = _dot(a1, w2r[...]) + b2r[...][0:1]
        _acc_stats(outr, z2, m8r[...][:, 4:5], pl.program_id(0))

    E = msg8.shape[0]
    ins = [msg8] + ([xg] if has_x else []) + [w8] + ([wx] if has_x else [])         + [p1, w2, b2]
    specs = [_e_spec(R, 8)] + ([_e_spec(R, 128)] if has_x else [])         + [_w_spec(8, F)] + ([_w_spec(128, F)] if has_x else [])         + [_w_spec(8, F), _w_spec(F, G), _w_spec(8, G)]
    return pl.pallas_call(
        body, grid=(E // R,), in_specs=specs,
        out_specs=pl.BlockSpec((8, 128), lambda i: (0, 0)),
        out_shape=jax.ShapeDtypeStruct((8, 128), _F32),
    )(*ins)


def _zmask(msg8, w8, p1, w2, b2, p2, w3, b3, R, F, G, H, xg=None, wx=None):
    has_x = xg is not None

    def body(*refs):
        if has_x:
            m8r, xgr, w8r, wxr, p1r, w2r, b2r, p2r, w3r, b3r, outr = refs
            z1 = _dot(m8r[...], w8r[...]) + _dot(xgr[...], wxr[...])
        else:
            m8r, w8r, p1r, w2r, b2r, p2r, w3r, b3r, outr = refs
            z1 = _dot(m8r[...], w8r[...])
        p1v = p1r[...]
        a1 = jax.nn.relu(z1 * p1v[0:1] + p1v[1:2])
        z2 = _dot(a1, w2r[...]) + b2r[...][0:1]
        p2v = p2r[...]
        a2 = jax.nn.relu(z2 * p2v[0:1] + p2v[1:2])
        z3 = _dot(a2, w3r[...]) + b3r[...][0:1]
        vm = m8r[...][:, 4:5]
        outr[...] = jnp.where(vm > 0, z3, -jnp.inf)

    E = msg8.shape[0]
    ins = [msg8] + ([xg] if has_x else []) + [w8] + ([wx] if has_x else [])         + [p1, w2, b2, p2, w3, b3]
    specs = [_e_spec(R, 8)] + ([_e_spec(R, 128)] if has_x else [])         + [_w_spec(8, F)] + ([_w_spec(128, F)] if has_x else [])         + [_w_spec(8, F), _w_spec(F, G), _w_spec(8, G),
           _w_spec(8, G), _w_spec(G, H), _w_spec(8, H)]
    return pl.pallas_call(
        body, grid=(E // R,), in_specs=specs,
        out_specs=pl.BlockSpec((R, H), lambda i: (i, 0)),
        out_shape=jax.ShapeDtypeStruct((E, H), _F32),
    )(*ins)


def _seg_max(z3, R, H):
    NP_ = z3.shape[0]

    def body(zr, outr):
        xm = zr[...].max(axis=1)
        outr[...] = jnp.where(xm > -jnp.inf, xm, 0.0)

    return pl.pallas_call(
        body, grid=(NP_ // R,),
        in_specs=[pl.BlockSpec((R, MAXNN, H), lambda i: (i, 0, 0))],
        out_specs=pl.BlockSpec((R, H), lambda i: (i, 0)),
        out_shape=jax.ShapeDtypeStruct((NP_, H), _F32),
    )(z3)


def _rowmax(a):
    w = a.shape[1]
    while w > 1:
        w //= 2
        a = jnp.maximum(a[:, :w], a[:, w:2 * w])
    return a


def _rowsum(a):
    w = a.shape[1]
    while w > 1:
        w //= 2
        a = a[:, :w] + a[:, w:2 * w]
    return a


def _global_stage(x2, aux, w31x, w31d, g31, w32, b32, g32, w33, b33,
                  h1, b1, h2, b2, h3, b3):
    def body(x2r, auxr, w31xr, w31dr, g31r, w32r, b32r, g32r, w33r, b33r,
             h1r, b1r, h2r, b2r, h3r, b3r, outr):
        x2v = x2r[...]
        aux = auxr[...]
        vm = aux[:, 3:4]
        n = jnp.maximum(vm.sum(), 1.0)

        def bn_relu(z, gb):
            m = (z * vm).sum(0) / n
            d = z - m
            v = (d * d * vm).sum(0) / n
            return jax.nn.relu(d / jnp.sqrt(v + 1e-5) * gb[0:1] + gb[1:2])

        wd = w31dr[...]
        D = wd.shape[1]
        z = _dot(x2v, w31xr[...])
        for cc in range(3):
            z = z + (jnp.broadcast_to(aux[:, cc:cc + 1], z.shape)
                     * jnp.broadcast_to(wd[cc:cc + 1], z.shape))
        z = z + wd[3:4]
        a = bn_relu(z, g31r[...])
        z = _dot(a, w32r[...]) + b32r[...][0:1]
        a = bn_relu(z, g32r[...])
        h = _dot(a, w33r[...]) + b33r[...][0:1]

        HD = h.shape[1]
        xgc = jnp.full((NCLOUD, HD), -jnp.inf)
        rowi = lax.broadcasted_iota(_I32, (NCLOUD, HD), 0)
        for c in range(NCLOUD):
            mk = aux[:, 16 + c:17 + c] > 0
            seg = jnp.where(mk, h, -jnp.inf).max(axis=0)
            xgc = jnp.where(rowi == c, jnp.broadcast_to(seg, (NCLOUD, HD)),
                            xgc)
        xgc = jnp.where(xgc > -jnp.inf, xgc, 0.0)

        a = jax.nn.relu(_dot(xgc, h1r[...]) + b1r[...][0:1])
        a = jax.nn.relu(_dot(a, h2r[...]) + b2r[...][0:1])
        o = _dot(a, h3r[...]) + b3r[...][0:1]
        col = lax.broadcasted_iota(_I32, (NCLOUD, 128), 1)
        om = jnp.where(col < 10, o, -jnp.inf)
        mx = _rowmax(om)
        ls = mx + jnp.log(_rowsum(jnp.exp(om - jnp.broadcast_to(mx, om.shape))))
        outr[...] = o - jnp.broadcast_to(ls, o.shape)

    full = lambda s_: pl.BlockSpec(s_, lambda: tuple(0 for _ in s_))
    return pl.pallas_call(
        body,
        in_specs=[full((N2P, 256)), full((N2P, 128)),
                  full((256, 256)), full((8, 256)), full((8, 256)),
                  full((256, 512)), full((8, 512)), full((8, 512)),
                  full((512, 1024)), full((8, 1024)),
                  full((1024, 512)), full((8, 512)),
                  full((512, 256)), full((8, 256)),
                  full((256, 128)), full((8, 128))],
        out_specs=full((NCLOUD, 128)),
        out_shape=jax.ShapeDtypeStruct((NCLOUD, 128), _F32),
    )(x2, aux, w31x, w31d, g31, w32, b32, g32, w33, b33,
      h1, b1, h2, b2, h3, b3)


def _pad_rows(v, d):
    out = jnp.zeros((8, d), _F32)
    return out.at[0, :v.shape[0]].set(v)


def _pack2(a, b, d):
    out = jnp.zeros((8, d), _F32)
    return out.at[0, :a.shape[0]].set(a).at[1, :b.shape[0]].set(b)


def _stats_to_scale(stats, d, g, bt):
    S = stats[0, :d]
    Q = stats[1, :d]
    n = jnp.maximum(stats[2, 0], 1.0)
    m = S / n
    v = Q / n - m * m
    s = g / jnp.sqrt(v + 1e-5)
    return _pack2(s, bt - m * s, d)


def _first_layer_pack(W, b, d):
    out = jnp.zeros((8, d), _F32)
    out = out.at[0:3, :].set(W)
    return out.at[3, :].set(b)


def kernel(pos, batch, mlp1, mlp2, mlp3, head):
    pos = pos.astype(_F32)
    batch = batch.astype(_I32)

    counts = jnp.zeros((NCLOUD,), _I32).at[batch].add(1)
    starts = jnp.concatenate([jnp.zeros((1,), _I32),
                              jnp.cumsum(counts)[:-1].astype(_I32)])
    ranks = jnp.arange(NPTS, dtype=_I32) - starts[batch]
    sel1 = (ranks % 2) == 0
    idx1, = jnp.nonzero(sel1, size=N1, fill_value=0)
    nv1 = sel1.sum()
    idx1 = jnp.pad(idx1.astype(_I32), (0, N1P - N1))
    v1 = jnp.arange(N1P) < nv1
    b1 = jnp.where(v1, batch[idx1], 0)
    pos1 = pos[idx1]
    cst1 = jnp.where(v1, starts[b1], 0)
    cln1 = jnp.where(v1, counts[b1], 0)

    counts1 = jnp.zeros((NCLOUD,), _I32).at[b1].add(v1.astype(_I32))
    starts1 = jnp.concatenate([jnp.zeros((1,), _I32),
                               jnp.cumsum(counts1)[:-1].astype(_I32)])
    ranks1 = jnp.arange(N1P, dtype=_I32) - starts1[b1]
    sel2 = jnp.logical_and((ranks1 % 4) == 0, v1)
    idx2, = jnp.nonzero(sel2, size=N2, fill_value=0)
    nv2 = sel2.sum()
    idx2 = jnp.pad(idx2.astype(_I32), (0, N2P - N2))
    v2 = jnp.arange(N2P) < nv2
    b2 = jnp.where(v2, b1[idx2], 0)
    pos2 = pos1[idx2]
    cst2 = jnp.where(v2, starts1[b2], 0)
    cln2 = jnp.where(v2, counts1[b2], 0)

    p1x = pos1[:, 0].copy()
    p1y = pos1[:, 1].copy()
    p1z = pos1[:, 2].copy()

    dx1, dy1, dz1, cnt1 = _search_l1(
        pos[:, 0].copy(), pos[:, 1].copy(), pos[:, 2].copy(),
        p1x, p1y, p1z, cst1, cln1)
    vm1 = (jnp.arange(MAXNN)[None, :] < cnt1[:, None]).astype(_F32)

    dx2, dy2, dz2, nbr2, cnt2 = _search_l2(
        p1x, p1y, p1z,
        pos2[:, 0].copy(), pos2[:, 1].copy(), pos2[:, 2].copy(), cst2, cln2)
    vm2 = (jnp.arange(MAXNN)[None, :] < cnt2[:, None]).astype(_F32)

    E1P = N1P * MAXNN
    ones1 = jnp.ones((N1P, MAXNN), _F32)
    zz1 = jnp.zeros((N1P, MAXNN), _F32)
    msg8_1 = jnp.stack([dx1[:, :MAXNN], dy1[:, :MAXNN], dz1[:, :MAXNN],
                        ones1, vm1, zz1, zz1, zz1], axis=-1).reshape(E1P, 8)
    w8 = jnp.zeros((8, 64), _F32).at[0:3, :].set(mlp1[0]["W"])
    w8 = w8.at[3, :].set(mlp1[0]["b"])
    s1 = _stats1(msg8_1, w8, 8192, 64)
    p1 = _stats_to_scale(s1, 64, mlp1[0]["g"], mlp1[0]["bt"])
    w2 = mlp1[1]["W"]
    b2w = _pad_rows(mlp1[1]["b"], 64)
    s2 = _stats2(msg8_1, w8, p1, w2, b2w, 8192, 64, 64)
    p2 = _stats_to_scale(s2, 64, mlp1[1]["g"], mlp1[1]["bt"])
    z3m1 = _zmask(msg8_1, w8, p1, w2, b2w, p2, mlp1[2]["W"],
                  _pad_rows(mlp1[2]["b"], 128), 8192, 64, 64, 128)
    x1 = _seg_max(z3m1.reshape(N1P, MAXNN, 128), 256, 128)

    eidx = nbr2[:, :MAXNN].reshape(-1)
    xg = _gather_rows(x1, eidx)

    ones2 = jnp.ones((N2P, MAXNN), _F32)
    zz2 = jnp.zeros((N2P, MAXNN), _F32)
    geo8 = jnp.stack([dx2[:, :MAXNN], dy2[:, :MAXNN], dz2[:, :MAXNN],
                      ones2, vm2, zz2, zz2, zz2], axis=-1).reshape(E2P, 8)
    W21 = mlp2[0]["W"]
    wx = W21[:128, :]
    wd8 = jnp.zeros((8, 128), _F32).at[0:3, :].set(W21[128:, :])
    wd8 = wd8.at[3, :].set(mlp2[0]["b"])
    s1b = _stats1(geo8, wd8, 4096, 128, xg=xg, wx=wx)
    q1 = _stats_to_scale(s1b, 128, mlp2[0]["g"], mlp2[0]["bt"])
    w22 = mlp2[1]["W"]
    b22 = _pad_rows(mlp2[1]["b"], 128)
    s2b = _stats2(geo8, wd8, q1, w22, b22, 4096, 128, 128, xg=xg, wx=wx)
    q2 = _stats_to_scale(s2b, 128, mlp2[1]["g"], mlp2[1]["bt"])
    z3m2 = _zmask(geo8, wd8, q1, w22, b22, q2, mlp2[2]["W"],
                  _pad_rows(mlp2[2]["b"], 256), 4096, 128, 128, 256,
                  xg=xg, wx=wx)
    x2 = _seg_max(z3m2.reshape(N2P, MAXNN, 256), 128, 256)

    aux = jnp.zeros((N2P, 128), _F32)
    aux = aux.at[:, 0:3].set(pos2)
    aux = aux.at[:, 3].set(v2.astype(_F32))
    onehot = (b2[:, None] == jnp.arange(NCLOUD)[None, :])
    onehot = jnp.logical_and(onehot, v2[:, None]).astype(_F32)
    aux = aux.at[:, 16:32].set(onehot)

    W31 = mlp3[0]["W"]
    w31x = W31[:256, :]
    w31d = jnp.zeros((8, 256), _F32).at[0:3, :].set(W31[256:, :])
    w31d = w31d.at[3, :].set(mlp3[0]["b"])
    g31 = _pack2(mlp3[0]["g"], mlp3[0]["bt"], 256)
    g32 = _pack2(mlp3[1]["g"], mlp3[1]["bt"], 512)
    h3w = jnp.zeros((256, 128), _F32).at[:, :10].set(head[2]["W"])
    h3b = _pad_rows(head[2]["b"], 128)

    out = _global_stage(
        x2, aux, w31x, w31d, g31,
        mlp3[1]["W"], _pad_rows(mlp3[1]["b"], 512), g32,
        mlp3[2]["W"], _pad_rows(mlp3[2]["b"], 1024),
        head[0]["W"], _pad_rows(head[0]["b"], 512),
        head[1]["W"], _pad_rows(head[1]["b"], 256),
        h3w, h3b)
    return out[:, :10]
